# baseline (speedup 1.0000x reference)
import functools

import jax
import jax.numpy as jnp
from jax import lax
from jax.experimental import pallas as pl
from jax.experimental.pallas import tpu as pltpu
from jax.experimental.pallas import tpu_sc as plsc

LENGTH = 65536
ELEM = 256
NC = 1
NS = 16
L = 16
NW = NC * NS
OWN_PER_W = LENGTH // NW
NVEC = ELEM // L

SC_ROWS_PER_W = 128
R_TC = LENGTH - NW * SC_ROWS_PER_W

BR = 7936
NG = R_TC // BR

_mesh = plsc.VectorSubcoreMesh(
    core_axis_name="c", subcore_axis_name="s", num_cores=NC, num_subcores=NS
)


@functools.partial(
    pl.kernel,
    out_type=jax.ShapeDtypeStruct((NW, ELEM), jnp.float32),
    mesh=_mesh,
    scratch_types=[
        pltpu.VMEM((SC_ROWS_PER_W, ELEM), jnp.float32),
        pltpu.VMEM((L, ELEM), jnp.float32),
        pltpu.VMEM((1, ELEM), jnp.float32),
        pltpu.VMEM((L,), jnp.int32),
        pltpu.VMEM((1, ELEM), jnp.float32),
        pltpu.SemaphoreType.DMA,
        pltpu.SemaphoreType.DMA,
    ],
)
def _sc_partial_sums(buf_hbm, inp_hbm, iv_hbm, out_hbm,
                     slab_v, rowi_v, inp_v, iv_v, acc_v, sem, sem_g):
    wid = lax.axis_index("s") * NC + lax.axis_index("c")
    slab = R_TC + wid * SC_ROWS_PER_W

    slab_cp = pltpu.async_copy(
        buf_hbm.at[pl.ds(slab, SC_ROWS_PER_W)], slab_v, sem
    )

    pltpu.sync_copy(iv_hbm, iv_v)
    ivec = iv_v[...]
    pltpu.async_copy(buf_hbm.at[iv_v], rowi_v, sem_g).wait()
    pltpu.sync_copy(inp_hbm, inp_v)

    slab_cp.wait()

    def row_body(r, a):
        return tuple(a[j] + slab_v[r, pl.ds(j * L, L)] for j in range(NVEC))

    acc = lax.fori_loop(
        0, SC_ROWS_PER_W, row_body,
        tuple(jnp.zeros((L,), jnp.float32) for _ in range(NVEC)),
    )

    own = wid * OWN_PER_W
    owner = jnp.logical_and(ivec >= own, ivec < own + OWN_PER_W)
    w = jnp.where(owner, jnp.float32(1.0), jnp.float32(0.0))
    for j in range(NVEC):
        sl = pl.ds(j * L, L)
        acc_v[0, sl] = acc[j] + (inp_v[0, sl] - rowi_v[0, sl]) * w

    pltpu.sync_copy(acc_v, out_hbm.at[pl.ds(wid, 1)])


def _tc_reduce(x_ref, sc_ref, o_ref):
    g = pl.program_id(0)

    @pl.when(g == 0)
    def _init():
        o_ref[...] = jnp.zeros_like(o_ref)

    o_ref[...] += jnp.sum(x_ref[...], axis=0, keepdims=True)

    @pl.when(g == NG - 1)
    def _finish():
        o_ref[...] = (
            o_ref[...] + jnp.sum(sc_ref[...], axis=0, keepdims=True)
        ) * (1.0 / LENGTH)


_tc_fold = pl.pallas_call(
    _tc_reduce,
    grid=(NG,),
    in_specs=[
        pl.BlockSpec((BR, ELEM), lambda g: (g, 0)),
        pl.BlockSpec((NW, ELEM), lambda g: (0, 0)),
    ],
    out_specs=pl.BlockSpec((1, ELEM), lambda g: (0, 0)),
    out_shape=jax.ShapeDtypeStruct((1, ELEM), jnp.float32),
)


def kernel(inputs, buffer, index):
    i = (jnp.asarray(index, jnp.int32) + 1) % LENGTH
    iv = jnp.full((L,), i, dtype=jnp.int32)
    sc_partials = _sc_partial_sums(buffer, inputs.reshape(1, ELEM), iv)
    return _tc_fold(buffer, sc_partials)[0]


# speedup vs baseline: 1.0202x; 1.0202x over previous
import functools

import jax
import jax.numpy as jnp
from jax import lax
from jax.experimental import pallas as pl
from jax.experimental.pallas import tpu as pltpu
from jax.experimental.pallas import tpu_sc as plsc

LENGTH = 65536
ELEM = 256
NC = 1
NS = 16
L = 16
NW = NC * NS
OWN_PER_W = LENGTH // NW
NVEC = ELEM // L

SC_ROWS_PER_W = 0
R_TC = LENGTH - NW * SC_ROWS_PER_W

BR = 8192
NG = R_TC // BR

_mesh = plsc.VectorSubcoreMesh(
    core_axis_name="c", subcore_axis_name="s", num_cores=NC, num_subcores=NS
)


@functools.partial(
    pl.kernel,
    out_type=jax.ShapeDtypeStruct((NW, ELEM), jnp.float32),
    mesh=_mesh,
    scratch_types=[
        pltpu.VMEM((L, ELEM), jnp.float32),
        pltpu.VMEM((1, ELEM), jnp.float32),
        pltpu.VMEM((L,), jnp.int32),
        pltpu.VMEM((1, ELEM), jnp.float32),
        pltpu.SemaphoreType.DMA,
        pltpu.SemaphoreType.DMA,
    ],
)
def _sc_partial_sums(buf_hbm, inp_hbm, iv_hbm, out_hbm,
                     rowi_v, inp_v, iv_v, acc_v, sem, sem_g):
    wid = lax.axis_index("s") * NC + lax.axis_index("c")
    del sem

    pltpu.sync_copy(iv_hbm, iv_v)
    ivec = iv_v[...]
    pltpu.async_copy(buf_hbm.at[iv_v], rowi_v, sem_g).wait()
    pltpu.sync_copy(inp_hbm, inp_v)

    acc = tuple(jnp.zeros((L,), jnp.float32) for _ in range(NVEC))

    own = wid * OWN_PER_W
    owner = jnp.logical_and(ivec >= own, ivec < own + OWN_PER_W)
    w = jnp.where(owner, jnp.float32(1.0), jnp.float32(0.0))
    for j in range(NVEC):
        sl = pl.ds(j * L, L)
        acc_v[0, sl] = acc[j] + (inp_v[0, sl] - rowi_v[0, sl]) * w

    pltpu.sync_copy(acc_v, out_hbm.at[pl.ds(wid, 1)])


def _tc_reduce(x_ref, sc_ref, o_ref):
    g = pl.program_id(0)

    @pl.when(g == 0)
    def _init():
        o_ref[...] = jnp.zeros_like(o_ref)

    o_ref[...] += jnp.sum(x_ref[...], axis=0, keepdims=True)

    @pl.when(g == NG - 1)
    def _finish():
        o_ref[...] = (
            o_ref[...] + jnp.sum(sc_ref[...], axis=0, keepdims=True)
        ) * (1.0 / LENGTH)


_tc_fold = pl.pallas_call(
    _tc_reduce,
    grid=(NG,),
    in_specs=[
        pl.BlockSpec((BR, ELEM), lambda g: (g, 0)),
        pl.BlockSpec((NW, ELEM), lambda g: (0, 0)),
    ],
    out_specs=pl.BlockSpec((1, ELEM), lambda g: (0, 0)),
    out_shape=jax.ShapeDtypeStruct((1, ELEM), jnp.float32),
)


def kernel(inputs, buffer, index):
    i = (jnp.asarray(index, jnp.int32) + 1) % LENGTH
    iv = jnp.full((L,), i, dtype=jnp.int32)
    sc_partials = _sc_partial_sums(buffer, inputs.reshape(1, ELEM), iv)
    return _tc_fold(buffer, sc_partials)[0]


# speedup vs baseline: 1.0472x; 1.0265x over previous
import functools

import jax
import jax.numpy as jnp
from jax import lax
from jax.experimental import pallas as pl
from jax.experimental.pallas import tpu as pltpu
from jax.experimental.pallas import tpu_sc as plsc

LENGTH = 65536
ELEM = 256
NC = 1
NS = 16
L = 16
NW = NC * NS
OWN_PER_W = LENGTH // NW
NVEC = ELEM // L

SC_ROWS_PER_W = 0
R_TC = LENGTH - NW * SC_ROWS_PER_W

BR = 8192
NG = R_TC // BR

_mesh = plsc.VectorSubcoreMesh(
    core_axis_name="c", subcore_axis_name="s", num_cores=NC, num_subcores=NS
)


@functools.partial(
    pl.kernel,
    out_type=jax.ShapeDtypeStruct((NW, ELEM), jnp.float32),
    mesh=_mesh,
    scratch_types=[
        pltpu.VMEM((L, ELEM), jnp.float32),
        pltpu.VMEM((1, ELEM), jnp.float32),
        pltpu.VMEM((L,), jnp.int32),
        pltpu.VMEM((1, ELEM), jnp.float32),
        pltpu.SemaphoreType.DMA,
        pltpu.SemaphoreType.DMA,
    ],
)
def _sc_partial_sums(buf_hbm, inp_hbm, iv_hbm, out_hbm,
                     rowi_v, inp_v, iv_v, acc_v, sem, sem_g):
    wid = lax.axis_index("s") * NC + lax.axis_index("c")
    del sem

    pltpu.sync_copy(iv_hbm, iv_v)
    ivec = iv_v[...]
    pltpu.async_copy(buf_hbm.at[iv_v], rowi_v, sem_g).wait()
    pltpu.sync_copy(inp_hbm, inp_v)

    acc = tuple(jnp.zeros((L,), jnp.float32) for _ in range(NVEC))

    own = wid * OWN_PER_W
    owner = jnp.logical_and(ivec >= own, ivec < own + OWN_PER_W)
    w = jnp.where(owner, jnp.float32(1.0), jnp.float32(0.0))
    for j in range(NVEC):
        sl = pl.ds(j * L, L)
        acc_v[0, sl] = acc[j] + (inp_v[0, sl] - rowi_v[0, sl]) * w

    pltpu.sync_copy(acc_v, out_hbm.at[pl.ds(wid, 1)])


def _tc_reduce(x_ref, o_ref):
    g = pl.program_id(0)

    @pl.when(g == 0)
    def _init():
        o_ref[...] = jnp.zeros_like(o_ref)

    o_ref[...] += jnp.sum(x_ref[...], axis=0, keepdims=True)


_tc_colsum = pl.pallas_call(
    _tc_reduce,
    grid=(NG,),
    in_specs=[pl.BlockSpec((BR, ELEM), lambda g: (g, 0))],
    out_specs=pl.BlockSpec((1, ELEM), lambda g: (0, 0)),
    out_shape=jax.ShapeDtypeStruct((1, ELEM), jnp.float32),
)


def kernel(inputs, buffer, index):
    i = (jnp.asarray(index, jnp.int32) + 1) % LENGTH
    iv = jnp.full((L,), i, dtype=jnp.int32)
    tc_partial = _tc_colsum(buffer)
    sc_partials = _sc_partial_sums(buffer, inputs.reshape(1, ELEM), iv)
    return (tc_partial[0] + sc_partials.sum(axis=0)) * (1.0 / LENGTH)


# speedup vs baseline: 1.2559x; 1.1993x over previous
import functools

import jax
import jax.numpy as jnp
from jax import lax
from jax.experimental import pallas as pl
from jax.experimental.pallas import tpu as pltpu
from jax.experimental.pallas import tpu_sc as plsc

LENGTH = 65536
ELEM = 256
NC = 1
NS = 16
L = 16
NVEC = ELEM // L

BR = 8192
NG = LENGTH // BR

_mesh = plsc.VectorSubcoreMesh(
    core_axis_name="c", subcore_axis_name="s", num_cores=NC, num_subcores=NS
)


def _tc_reduce(x_ref, o_ref):
    @pl.when(pl.program_id(0) == 0)
    def _init():
        o_ref[...] = jnp.zeros_like(o_ref)

    o_ref[...] += jnp.sum(x_ref[...], axis=0, keepdims=True)


_tc_colsum = pl.pallas_call(
    _tc_reduce,
    grid=(NG,),
    in_specs=[pl.BlockSpec((BR, ELEM), lambda g: (g, 0))],
    out_specs=pl.BlockSpec((1, ELEM), lambda g: (0, 0)),
    out_shape=jax.ShapeDtypeStruct((1, ELEM), jnp.float32),
)


@functools.partial(
    pl.kernel,
    out_type=jax.ShapeDtypeStruct((1, ELEM), jnp.float32),
    mesh=_mesh,
    scratch_types=[
        pltpu.VMEM((L, ELEM), jnp.float32),
        pltpu.VMEM((1, ELEM), jnp.float32),
        pltpu.VMEM((1, ELEM), jnp.float32),
        pltpu.VMEM((L,), jnp.int32),
        pltpu.VMEM((1, ELEM), jnp.float32),
        pltpu.SemaphoreType.DMA,
    ],
)
def _sc_finish(buf_hbm, inp_hbm, tc_hbm, iv_hbm, out_hbm,
               rowi_v, inp_v, tc_v, iv_v, o_v, sem_g):
    wid = lax.axis_index("s") * NC + lax.axis_index("c")

    @pl.when(wid == 0)
    def _():
        pltpu.sync_copy(iv_hbm, iv_v)
        pltpu.async_copy(buf_hbm.at[iv_v], rowi_v, sem_g).wait()
        pltpu.sync_copy(inp_hbm, inp_v)
        pltpu.sync_copy(tc_hbm, tc_v)
        for j in range(NVEC):
            sl = pl.ds(j * L, L)
            o_v[0, sl] = (
                tc_v[0, sl] + inp_v[0, sl] - rowi_v[0, sl]
            ) * (1.0 / LENGTH)
        pltpu.sync_copy(o_v, out_hbm)


def kernel(inputs, buffer, index):
    i = (jnp.asarray(index, jnp.int32) + 1) % LENGTH
    iv = jnp.full((L,), i, dtype=jnp.int32)
    tc_partial = _tc_colsum(buffer)
    return _sc_finish(buffer, inputs.reshape(1, ELEM), tc_partial, iv)[0]


# speedup vs baseline: 1.2900x; 1.0272x over previous
import functools

import jax
import jax.numpy as jnp
from jax import lax
from jax.experimental import pallas as pl
from jax.experimental.pallas import tpu as pltpu
from jax.experimental.pallas import tpu_sc as plsc

LENGTH = 65536
ELEM = 256
NC = 1
NS = 16
L = 16
NVEC = ELEM // L

BR = 8192
NG = LENGTH // BR

_mesh = plsc.VectorSubcoreMesh(
    core_axis_name="c", subcore_axis_name="s", num_cores=NC, num_subcores=NS
)


def _tc_reduce(x_ref, o_ref):
    @pl.when(pl.program_id(0) == 0)
    def _init():
        o_ref[...] = jnp.zeros_like(o_ref)

    o_ref[...] += jnp.sum(x_ref[...], axis=0, keepdims=True)


_tc_colsum = pl.pallas_call(
    _tc_reduce,
    grid=(NG,),
    in_specs=[pl.BlockSpec((BR, ELEM), lambda g: (g, 0))],
    out_specs=pl.BlockSpec((1, ELEM), lambda g: (0, 0)),
    out_shape=jax.ShapeDtypeStruct((1, ELEM), jnp.float32),
)


@functools.partial(
    pl.kernel,
    out_type=jax.ShapeDtypeStruct((1, ELEM), jnp.float32),
    mesh=_mesh,
    scratch_types=[
        pltpu.VMEM((L, ELEM), jnp.float32),
        pltpu.VMEM((1, ELEM), jnp.float32),
        pltpu.VMEM((1, ELEM), jnp.float32),
        pltpu.VMEM((L,), jnp.int32),
        pltpu.VMEM((1, ELEM), jnp.float32),
        pltpu.SemaphoreType.DMA,
        pltpu.SemaphoreType.DMA,
        pltpu.SemaphoreType.DMA,
    ],
)
def _sc_finish(buf_hbm, inp_hbm, tc_hbm, iv_hbm, out_hbm,
               rowi_v, inp_v, tc_v, iv_v, o_v, sem_g, sem_i, sem_t):
    wid = lax.axis_index("s") * NC + lax.axis_index("c")

    @pl.when(wid == 0)
    def _():
        inp_cp = pltpu.async_copy(inp_hbm, inp_v, sem_i)
        tc_cp = pltpu.async_copy(tc_hbm, tc_v, sem_t)
        pltpu.sync_copy(iv_hbm, iv_v)
        pltpu.async_copy(buf_hbm.at[iv_v], rowi_v, sem_g).wait()
        inp_cp.wait()
        tc_cp.wait()
        for j in range(NVEC):
            sl = pl.ds(j * L, L)
            o_v[0, sl] = (
                tc_v[0, sl] + inp_v[0, sl] - rowi_v[0, sl]
            ) * (1.0 / LENGTH)
        pltpu.sync_copy(o_v, out_hbm)


def kernel(inputs, buffer, index):
    i = (jnp.asarray(index, jnp.int32) + 1) % LENGTH
    iv = jnp.full((L,), i, dtype=jnp.int32)
    tc_partial = _tc_colsum(buffer)
    return _sc_finish(buffer, inputs.reshape(1, ELEM), tc_partial, iv)[0]
